# Initial kernel scaffold; baseline (speedup 1.0000x reference)
import jax
import jax.numpy as jnp
from jax.experimental import pallas as pl


def kernel(indices, table):
    raise NotImplementedError("write your pallas kernel here")


# speedup vs baseline: 4.8075x; 4.8075x over previous
import functools

import jax
import jax.numpy as jnp
from jax import lax
from jax.experimental import pallas as pl
from jax.experimental.pallas import tpu as pltpu
from jax.experimental.pallas import tpu_sc as plsc

_EMB = 32
_NUM_WORKERS = 32
_CHUNK = 1024


@functools.partial(jax.jit, static_argnames=())
def _embedding_lookup(idx_flat, table):
    total = idx_flat.shape[0]
    per_worker = total // _NUM_WORKERS
    n_chunks = per_worker // _CHUNK
    mesh = plsc.VectorSubcoreMesh(core_axis_name="c", subcore_axis_name="s")

    @functools.partial(
        pl.kernel,
        mesh=mesh,
        out_type=jax.ShapeDtypeStruct((total, _EMB), jnp.float32),
        scratch_types=[
            pltpu.VMEM((_CHUNK,), jnp.int32),
            pltpu.VMEM((_CHUNK, _EMB), jnp.float32),
            pltpu.SemaphoreType.DMA,
        ],
        compiler_params=pltpu.CompilerParams(use_tc_tiling_on_sc=False),
    )
    def emb_kernel(idx_hbm, table_hbm, out_hbm, idx_v, rows_v, sem):
        wid = lax.axis_index("s") * 2 + lax.axis_index("c")
        base = wid * per_worker

        def body(i, _):
            off = base + i * _CHUNK
            pltpu.sync_copy(idx_hbm.at[pl.ds(off, _CHUNK)], idx_v)
            pltpu.async_copy(table_hbm.at[idx_v], rows_v, sem).wait()
            pltpu.sync_copy(rows_v, out_hbm.at[pl.ds(off, _CHUNK)])
            return ()

        lax.fori_loop(0, n_chunks, body, (), unroll=False)

    return emb_kernel(idx_flat, table)


def kernel(indices, table):
    idx_flat = indices.reshape(-1).astype(jnp.int32)
    out = _embedding_lookup(idx_flat, table)
    return out.reshape(indices.shape + (_EMB,))


# speedup vs baseline: 5.0316x; 1.0466x over previous
import functools

import jax
import jax.numpy as jnp
from jax import lax
from jax.experimental import pallas as pl
from jax.experimental.pallas import tpu as pltpu
from jax.experimental.pallas import tpu_sc as plsc

_EMB = 32
_NUM_WORKERS = 32
_CHUNK = 1024
_NBUF = 2


def _make_embedding_lookup(total):
    per_worker = total // _NUM_WORKERS
    n_chunks = per_worker // _CHUNK
    n_outer = n_chunks // _NBUF
    assert per_worker * _NUM_WORKERS == total
    assert n_outer * _NBUF == n_chunks
    mesh = plsc.VectorSubcoreMesh(core_axis_name="c", subcore_axis_name="s")

    scratch = (
        [pltpu.VMEM((_CHUNK,), jnp.int32) for _ in range(_NBUF)]
        + [pltpu.VMEM((_CHUNK, _EMB), jnp.float32) for _ in range(_NBUF)]
        + [pltpu.SemaphoreType.DMA for _ in range(3 * _NBUF)]
    )

    @functools.partial(
        pl.kernel,
        mesh=mesh,
        out_type=jax.ShapeDtypeStruct((total, _EMB), jnp.float32),
        scratch_types=scratch,
        compiler_params=pltpu.CompilerParams(use_tc_tiling_on_sc=False),
    )
    def emb_kernel(idx_hbm, table_hbm, out_hbm, *scr):
        idx_v = scr[:_NBUF]
        rows_v = scr[_NBUF : 2 * _NBUF]
        si = scr[2 * _NBUF : 3 * _NBUF]
        sg = scr[3 * _NBUF : 4 * _NBUF]
        so = scr[4 * _NBUF : 5 * _NBUF]

        wid = lax.axis_index("s") * 2 + lax.axis_index("c")
        base = wid * per_worker

        def sl(c):
            return pl.ds(base + c * _CHUNK, _CHUNK)

        def start_idx(c, b):
            pltpu.async_copy(idx_hbm.at[sl(c)], idx_v[b], si[b])

        def wait_idx(b):
            pltpu.make_async_copy(idx_hbm.at[sl(0)], idx_v[b], si[b]).wait()

        def start_gather(c, b):
            pltpu.async_copy(table_hbm.at[idx_v[b]], rows_v[b], sg[b])

        def wait_gather(b):
            pltpu.make_async_copy(
                table_hbm.at[idx_v[b]], rows_v[b], sg[b]
            ).wait()

        def start_out(c, b):
            pltpu.async_copy(rows_v[b], out_hbm.at[sl(c)], so[b])

        def wait_out(b):
            pltpu.make_async_copy(rows_v[b], out_hbm.at[sl(0)], so[b]).wait()

        def step(c, b, prefetch, launch_next, wait_prev_out):
            wait_gather(b)
            start_out(c, b)
            if prefetch:
                start_idx(c + _NBUF, b)
            if launch_next:
                b1 = (b + 1) % _NBUF
                wait_idx(b1)
                if wait_prev_out:
                    wait_out(b1)
                start_gather(c + 1, b1)

        pltpu.sync_copy(idx_hbm.at[sl(0)], idx_v[0])
        start_gather(0, 0)
        for b in range(1, _NBUF):
            start_idx(b, b)

        for b in range(_NBUF):
            step(b, b, prefetch=True, launch_next=True,
                 wait_prev_out=(b == _NBUF - 1))

        def outer(g, _):
            c0 = g * _NBUF
            for b in range(_NBUF):
                step(c0 + b, b, prefetch=True, launch_next=True,
                     wait_prev_out=True)
            return ()

        lax.fori_loop(1, n_outer - 1, outer, (), unroll=False)

        c0 = (n_outer - 1) * _NBUF
        for b in range(_NBUF):
            step(c0 + b, b, prefetch=False,
                 launch_next=(b != _NBUF - 1), wait_prev_out=True)

        for b in range(_NBUF):
            wait_out(b)

    return emb_kernel


def kernel(indices, table):
    idx_flat = indices.reshape(-1).astype(jnp.int32)
    out = _make_embedding_lookup(idx_flat.shape[0])(idx_flat, table)
    return out.reshape(indices.shape + (_EMB,))
